# baseline (speedup 1.0000x reference)
import functools

import jax
import jax.numpy as jnp
from jax import lax
from jax.experimental import pallas as pl
from jax.experimental.pallas import tpu as pltpu
from jax.experimental.pallas import tpu_sc as plsc

NUM_CLASSES = 80
K = 300
KPAD = 384
CH = 128
NCHUNK = 12500
HALFR = 6250
HPAD = 6272
NCPAD = 2 * HPAD
WIN = 16384
NWIN = 48
TROWS = HALFR - NWIN * 128
MSEL = 384
CCAP = 512
NQ = 20000
NEG_INF = float("-inf")
IMIN = -0x80000000


def _iota16():
    return lax.broadcasted_iota(jnp.int32, (16,), 0)


def _splat_i(x):
    return jnp.zeros((16,), jnp.int32) + x


def _splat_f(x):
    return jnp.zeros((16,), jnp.float32) + x


def _float_of_key(k):
    b = jnp.where(k < 0, k ^ jnp.int32(0x7FFFFFFF), k)
    return lax.bitcast_convert_type(b, jnp.float32)


def _sc_body(logits1_hbm, logits_hbm, boxes_hbm,
             labels_hbm, oboxes_hbm, scores_hbm,
             win_v, maxloc_v, maxful_v, ids_v, gath_v,
             candv_v, candi_v, outv_v, outi_v,
             labels_v, scores_v, qidx4_v, boxg_v, obox_v,
             shared_max, sem):
    cid_ = lax.axis_index("c")
    sid = lax.axis_index("s")
    b = cid_ * 8 + lax.rem(sid, 8)
    h = sid // 8
    base_el = b * (NCHUNK * CH) + h * (HALFR * CH)
    lanes = _iota16()
    ninf16 = jnp.full((16,), NEG_INF, jnp.float32)

    def block_max(rbase):
        def j_body(j, acc):
            for dj in range(8):
                v = plsc.load_gather(win_v, [rbase + (j * 8 + dj)])
                acc = jnp.maximum(acc, v)
            return acc

        return lax.fori_loop(0, CH // 8, j_body, ninf16)

    def win_body(w, _):
        pltpu.sync_copy(logits1_hbm.at[pl.ds(base_el + w * WIN, WIN)], win_v)

        def blk_body(rb, _):
            acc = block_max((rb * 16 + lanes) * CH)
            maxloc_v[pl.ds(w * 128 + rb * 16, 16)] = acc
            return 0

        lax.fori_loop(0, 8, blk_body, 0)
        return 0

    lax.fori_loop(0, NWIN, win_body, 0)

    pltpu.sync_copy(logits1_hbm.at[pl.ds(base_el + NWIN * WIN, TROWS * CH)],
                    win_v.at[pl.ds(0, TROWS * CH)])
    for rb in range(8):
        if rb == 7:
            acc = ninf16
        else:
            rl = rb * 16 + lanes
            acc = block_max(jnp.minimum(rl, TROWS - 1) * CH)
            if rb == 6:
                acc = jnp.where(rl < TROWS, acc, ninf16)
        maxloc_v[pl.ds(NWIN * 128 + rb * 16, 16)] = acc

    srow = lax.rem(sid, 8) * NCPAD
    pltpu.sync_copy(maxloc_v, shared_max.at[pl.ds(srow + h * HPAD, HPAD)])
    plsc.subcore_barrier()

    @pl.when(h == 0)
    def _phase2():
        pltpu.sync_copy(shared_max.at[pl.ds(srow, NCPAD)], maxful_v)

        def count_ge(tf):
            def cnt_body(i, c):
                for di in range(2):
                    m = maxful_v[pl.ds((i * 2 + di) * 16, 16)]
                    c = c + jnp.where(m >= tf, 1, 0).astype(jnp.int32)
                return c

            cvec = lax.fori_loop(0, NCPAD // 32, cnt_body,
                                 jnp.zeros((16,), jnp.int32))
            return jnp.sum(cvec)

        def bs_body(_, carry):
            lo, hi = carry
            mid = (lo >> 1) + (hi >> 1) + (lo & hi & 1)
            big = count_ge(_float_of_key(mid)) >= K
            return (jnp.where(big, mid, lo), jnp.where(big, hi, mid))

        lo0 = jnp.int32(-0x7F800001)
        hi0 = jnp.int32(0x7F800000)
        t_key, _ = lax.fori_loop(0, 32, bs_body, (lo0, hi0))
        t_f = _float_of_key(t_key)

        def fill_ids(i, _):
            ids_v[pl.ds(i * 16, 16)] = b * NCHUNK + i * 16 + lanes
            return 0

        lax.fori_loop(0, MSEL // 16, fill_ids, 0)

        def cmp_body(i, off):
            m = maxful_v[pl.ds(i * 16, 16)]
            selm = m >= t_f
            e = i * 16 + lanes
            c = jnp.where(e < HPAD, e, e - (HPAD - HALFR))
            gid = b * NCHUNK + c
            plsc.store_compressed(ids_v.at[pl.ds(off, 16)], gid, mask=selm)
            return off + jnp.sum(jnp.where(selm, 1, 0).astype(jnp.int32))

        s_cnt = lax.fori_loop(0, NCPAD // 16, cmp_body, jnp.int32(0))
        s_use = jnp.minimum(s_cnt, MSEL)

        for g in range(MSEL // 128):
            pltpu.async_copy(
                logits_hbm.at[ids_v.at[pl.ds(g * 128, 128)]],
                gath_v.at[pl.ds(g * 128, 128)], sem).wait()

        def fill_cand(i, _):
            candv_v[pl.ds(i * 16, 16)] = ninf16
            candi_v[pl.ds(i * 16, 16)] = jnp.zeros((16,), jnp.int32)
            return 0

        lax.fori_loop(0, (CCAP + 16) // 16, fill_cand, 0)

        def cand_body(p, ce):
            idvec = ids_v[pl.ds((p // 16) * 16, 16)]
            cid = jnp.max(jnp.where(lanes == lax.rem(p, 16), idvec, IMIN))
            chunk_off = (cid - b * NCHUNK) * CH
            row = gath_v.at[p]
            for c0 in range(0, CH, 16):
                v = row[pl.ds(c0, 16)]
                selm = (v >= t_f) & (ce < CCAP)
                fidx = chunk_off + c0 + lanes
                plsc.store_compressed(candv_v.at[pl.ds(ce, 16)], v, mask=selm)
                plsc.store_compressed(candi_v.at[pl.ds(ce, 16)], fidx,
                                      mask=selm)
                ce = ce + jnp.sum(jnp.where(selm, 1, 0).astype(jnp.int32))
            return ce

        ce_cnt = lax.fori_loop(0, s_use, cand_body, jnp.int32(0))
        n_cvr = (jnp.minimum(ce_cnt, CCAP) + 15) // 16

        def fill_out(i, _):
            outv_v[pl.ds(i * 16, 16)] = jnp.zeros((16,), jnp.float32)
            outi_v[pl.ds(i * 16, 16)] = jnp.zeros((16,), jnp.int32)
            return 0

        lax.fori_loop(0, KPAD // 16, fill_out, 0)

        def rank_blk(ib, _):
            vals = candv_v[pl.ds(ib * 16, 16)]
            idxs = candi_v[pl.ds(ib * 16, 16)]

            def rank_lane(l, _):
                lsel = lanes == l
                sv = jnp.max(jnp.where(lsel, vals, NEG_INF))
                si = jnp.max(jnp.where(lsel, idxs, IMIN))

                def cnt_body(jb, c):
                    vj = candv_v[pl.ds(jb * 16, 16)]
                    ij = candi_v[pl.ds(jb * 16, 16)]
                    beat = (vj > sv) | ((vj == sv) & (ij < si))
                    return c + jnp.where(beat, 1, 0).astype(jnp.int32)

                rank = jnp.sum(lax.fori_loop(0, n_cvr, cnt_body,
                                             jnp.zeros((16,), jnp.int32)))
                okm = lsel & (rank < K) & (sv > NEG_INF)
                ridx = _splat_i(rank)
                plsc.store_scatter(outv_v, [ridx], _splat_f(sv), mask=okm)
                plsc.store_scatter(outi_v, [ridx], _splat_i(si), mask=okm)
                return 0

            lax.fori_loop(0, 16, rank_lane, 0)
            return 0

        lax.fori_loop(0, n_cvr, rank_blk, 0)

        def dec_body(i, _):
            r = i * 16 + lanes
            fl = outi_v[pl.ds(i * 16, 16)]
            q = fl // NUM_CLASSES
            labels_v[pl.ds(i * 16, 16)] = fl - q * NUM_CLASSES
            brow4 = (b * NQ + q) * 4
            plsc.store_scatter(qidx4_v, [r * 4], brow4)
            plsc.store_scatter(qidx4_v, [r * 4 + 1], brow4 + 1)
            plsc.store_scatter(qidx4_v, [r * 4 + 2], brow4 + 2)
            plsc.store_scatter(qidx4_v, [r * 4 + 3], brow4 + 3)
            sv = outv_v[pl.ds(i * 16, 16)]
            scores_v[pl.ds(i * 16, 16)] = 1.0 / (1.0 + jnp.exp(-sv))
            return 0

        lax.fori_loop(0, KPAD // 16, dec_body, 0)

        pltpu.async_copy(boxes_hbm.at[qidx4_v], boxg_v, sem).wait()

        def box_body(i, _):
            r4 = (i * 16 + lanes) * 4
            cx = plsc.load_gather(boxg_v, [r4]) * 640.0
            cy = plsc.load_gather(boxg_v, [r4 + 1]) * 640.0
            w_ = plsc.load_gather(boxg_v, [r4 + 2]) * 640.0
            h_ = plsc.load_gather(boxg_v, [r4 + 3]) * 640.0
            plsc.store_scatter(obox_v, [r4], cx - 0.5 * w_)
            plsc.store_scatter(obox_v, [r4 + 1], cy - 0.5 * h_)
            plsc.store_scatter(obox_v, [r4 + 2], cx + 0.5 * w_)
            plsc.store_scatter(obox_v, [r4 + 3], cy + 0.5 * h_)
            return 0

        lax.fori_loop(0, KPAD // 16, box_body, 0)

        pltpu.sync_copy(labels_v, labels_hbm.at[b])
        pltpu.sync_copy(obox_v, oboxes_hbm.at[b])
        pltpu.sync_copy(scores_v, scores_hbm.at[b])


def kernel(samples, pred_logits, pred_boxes):
    B = pred_logits.shape[0]
    zero = lax.optimization_barrier(jnp.zeros((), jnp.float32))
    logits_lin = pred_logits + zero
    boxes_lin = pred_boxes + zero
    logits1d = logits_lin.reshape(B * NCHUNK * CH)
    logits2d = logits_lin.reshape(B * NCHUNK, CH)
    boxes1d = boxes_lin.reshape(B * NQ * 4)

    mesh = plsc.VectorSubcoreMesh(core_axis_name="c", subcore_axis_name="s")
    run = functools.partial(
        pl.kernel, mesh=mesh,
        out_type=[
            jax.ShapeDtypeStruct((B, KPAD), jnp.int32),
            jax.ShapeDtypeStruct((B, KPAD * 4), jnp.float32),
            jax.ShapeDtypeStruct((B, KPAD), jnp.float32),
        ],
        scratch_types=[
            pltpu.VMEM((WIN,), jnp.float32),
            pltpu.VMEM((HPAD,), jnp.float32),
            pltpu.VMEM((NCPAD,), jnp.float32),
            pltpu.VMEM((NCPAD + 16,), jnp.int32),
            pltpu.VMEM((MSEL, CH), jnp.float32),
            pltpu.VMEM((CCAP + 16,), jnp.float32),
            pltpu.VMEM((CCAP + 16,), jnp.int32),
            pltpu.VMEM((KPAD,), jnp.float32),
            pltpu.VMEM((KPAD,), jnp.int32),
            pltpu.VMEM((KPAD,), jnp.int32),
            pltpu.VMEM((KPAD,), jnp.float32),
            pltpu.VMEM((KPAD * 4,), jnp.int32),
            pltpu.VMEM((KPAD * 4,), jnp.float32),
            pltpu.VMEM((KPAD * 4,), jnp.float32),
            pltpu.VMEM_SHARED((8 * NCPAD,), jnp.float32),
            pltpu.SemaphoreType.DMA,
        ],
        compiler_params=pltpu.CompilerParams(needs_layout_passes=False),
    )(_sc_body)
    labels, oboxes, scores = run(logits1d, logits2d, boxes1d)
    return (labels[:, :K],
            oboxes.reshape(B, KPAD, 4)[:, :K],
            scores[:, :K])


# speedup vs baseline: 1.0917x; 1.0917x over previous
import functools

import jax
import jax.numpy as jnp
from jax import lax
from jax.experimental import pallas as pl
from jax.experimental.pallas import tpu as pltpu
from jax.experimental.pallas import tpu_sc as plsc

NUM_CLASSES = 80
K = 300
KPAD = 384
CH = 128
NCHUNK = 12500
HALFR = 6250
HPAD = 6272
NCPAD = 2 * HPAD
WIN = 16384
NWIN = 48
TROWS = HALFR - NWIN * 128
MSEL = 512
CCAP = 640
NSUP = NCPAD // 16
C2CAP = 320
NQ = 20000
NEG_INF = float("-inf")
IMIN = -0x80000000


def _iota16():
    return lax.broadcasted_iota(jnp.int32, (16,), 0)


def _splat_i(x):
    return jnp.zeros((16,), jnp.int32) + x


def _splat_f(x):
    return jnp.zeros((16,), jnp.float32) + x


def _float_of_key(k):
    b = jnp.where(k < 0, k ^ jnp.int32(0x7FFFFFFF), k)
    return lax.bitcast_convert_type(b, jnp.float32)


def _sc_body(logits1_hbm, logits_hbm, boxes_hbm,
             labels_hbm, oboxes_hbm, scores_hbm,
             win_v, maxloc_v, maxful_v, sup_v, ids_v, gath_v,
             candv_v, candi_v, c2v_v, c2i_v, outv_v, outi_v,
             labels_v, scores_v, qidx4_v, boxg_v, obox_v,
             shared_max, sem):
    cid_ = lax.axis_index("c")
    sid = lax.axis_index("s")
    b = cid_ * 8 + lax.rem(sid, 8)
    h = sid // 8
    base_el = b * (NCHUNK * CH) + h * (HALFR * CH)
    lanes = _iota16()
    ninf16 = jnp.full((16,), NEG_INF, jnp.float32)

    def block_max(rbase):
        def j_body(j, accs):
            a0, a1, a2, a3 = accs
            jb = j * 8
            a0 = jnp.maximum(a0, plsc.load_gather(win_v, [rbase + jb]))
            a1 = jnp.maximum(a1, plsc.load_gather(win_v, [rbase + jb + 1]))
            a2 = jnp.maximum(a2, plsc.load_gather(win_v, [rbase + jb + 2]))
            a3 = jnp.maximum(a3, plsc.load_gather(win_v, [rbase + jb + 3]))
            a0 = jnp.maximum(a0, plsc.load_gather(win_v, [rbase + jb + 4]))
            a1 = jnp.maximum(a1, plsc.load_gather(win_v, [rbase + jb + 5]))
            a2 = jnp.maximum(a2, plsc.load_gather(win_v, [rbase + jb + 6]))
            a3 = jnp.maximum(a3, plsc.load_gather(win_v, [rbase + jb + 7]))
            return (a0, a1, a2, a3)

        a0, a1, a2, a3 = lax.fori_loop(0, CH // 8, j_body,
                                       (ninf16, ninf16, ninf16, ninf16))
        return jnp.maximum(jnp.maximum(a0, a1), jnp.maximum(a2, a3))

    def win_body(w, _):
        pltpu.sync_copy(logits1_hbm.at[pl.ds(base_el + w * WIN, WIN)], win_v)

        def blk_body(rb, _):
            acc = block_max((rb * 16 + lanes) * CH)
            maxloc_v[pl.ds(w * 128 + rb * 16, 16)] = acc
            return 0

        lax.fori_loop(0, 8, blk_body, 0)
        return 0

    lax.fori_loop(0, NWIN, win_body, 0)

    pltpu.sync_copy(logits1_hbm.at[pl.ds(base_el + NWIN * WIN, TROWS * CH)],
                    win_v.at[pl.ds(0, TROWS * CH)])
    for rb in range(8):
        if rb == 7:
            acc = ninf16
        else:
            rl = rb * 16 + lanes
            acc = block_max(jnp.minimum(rl, TROWS - 1) * CH)
            if rb == 6:
                acc = jnp.where(rl < TROWS, acc, ninf16)
        maxloc_v[pl.ds(NWIN * 128 + rb * 16, 16)] = acc

    srow = lax.rem(sid, 8) * NCPAD
    pltpu.sync_copy(maxloc_v, shared_max.at[pl.ds(srow + h * HPAD, HPAD)])
    plsc.subcore_barrier()

    @pl.when(h == 0)
    def _phase2():
        pltpu.sync_copy(shared_max.at[pl.ds(srow, NCPAD)], maxful_v)

        def sup_body(i, _):
            sbase = (i * 16 + lanes) * 16
            a0, a1 = ninf16, ninf16
            for j in range(0, 16, 2):
                a0 = jnp.maximum(a0, plsc.load_gather(maxful_v, [sbase + j]))
                a1 = jnp.maximum(a1, plsc.load_gather(maxful_v, [sbase + j + 1]))
            sup_v[pl.ds(i * 16, 16)] = jnp.maximum(a0, a1)
            return 0

        lax.fori_loop(0, NSUP // 16, sup_body, 0)

        def count_sup(tf):
            cvec = jnp.zeros((16,), jnp.int32)
            for i in range(NSUP // 16):
                m = sup_v[pl.ds(i * 16, 16)]
                cvec = cvec + jnp.where(m >= tf, 1, 0).astype(jnp.int32)
            return jnp.sum(cvec)

        def bs_body(_, carry):
            lo, hi = carry
            mid = (lo >> 1) + (hi >> 1) + (lo & hi & 1)
            big = count_sup(_float_of_key(mid)) >= K
            return (jnp.where(big, mid, lo), jnp.where(big, hi, mid))

        lo0 = jnp.int32(-0x7F800001)
        hi0 = jnp.int32(0x7F800000)
        t_key, _ = lax.fori_loop(0, 32, bs_body, (lo0, hi0))
        t_f = _float_of_key(t_key)

        def fill_ids(i, _):
            ids_v[pl.ds(i * 16, 16)] = b * NCHUNK + i * 16 + lanes
            return 0

        lax.fori_loop(0, MSEL // 16, fill_ids, 0)

        def cmp_body(i, off):
            m = maxful_v[pl.ds(i * 16, 16)]
            selm = m >= t_f
            e = i * 16 + lanes
            c = jnp.where(e < HPAD, e, e - (HPAD - HALFR))
            gid = b * NCHUNK + c
            plsc.store_compressed(ids_v.at[pl.ds(off, 16)], gid, mask=selm)
            return off + jnp.sum(jnp.where(selm, 1, 0).astype(jnp.int32))

        s_cnt = lax.fori_loop(0, NCPAD // 16, cmp_body, jnp.int32(0))
        s_use = jnp.minimum(s_cnt, MSEL)

        for g in range(MSEL // 128):
            pltpu.async_copy(
                logits_hbm.at[ids_v.at[pl.ds(g * 128, 128)]],
                gath_v.at[pl.ds(g * 128, 128)], sem).wait()

        def fill_cand(i, _):
            candv_v[pl.ds(i * 16, 16)] = ninf16
            candi_v[pl.ds(i * 16, 16)] = jnp.zeros((16,), jnp.int32)
            return 0

        lax.fori_loop(0, (CCAP + 16) // 16, fill_cand, 0)

        def cand_body(p, ce):
            idvec = ids_v[pl.ds((p // 16) * 16, 16)]
            cid = jnp.max(jnp.where(lanes == lax.rem(p, 16), idvec, IMIN))
            chunk_off = (cid - b * NCHUNK) * CH
            row = gath_v.at[p]
            for c0 in range(0, CH, 16):
                v = row[pl.ds(c0, 16)]
                selm = (v >= t_f) & (ce < CCAP)
                fidx = chunk_off + c0 + lanes
                plsc.store_compressed(candv_v.at[pl.ds(ce, 16)], v, mask=selm)
                plsc.store_compressed(candi_v.at[pl.ds(ce, 16)], fidx,
                                      mask=selm)
                ce = ce + jnp.sum(jnp.where(selm, 1, 0).astype(jnp.int32))
            return ce

        ce_cnt = lax.fori_loop(0, s_use, cand_body, jnp.int32(0))
        n_cvr = (jnp.minimum(ce_cnt, CCAP) + 15) // 16

        def count_cand(tf):
            def cc_body(i, c):
                v = candv_v[pl.ds(i * 16, 16)]
                return c + jnp.where(v >= tf, 1, 0).astype(jnp.int32)

            return jnp.sum(lax.fori_loop(0, n_cvr, cc_body,
                                         jnp.zeros((16,), jnp.int32)))

        def bs2_body(_, carry):
            lo, hi = carry
            mid = (lo >> 1) + (hi >> 1) + (lo & hi & 1)
            big = count_cand(_float_of_key(mid)) >= K
            return (jnp.where(big, mid, lo), jnp.where(big, hi, mid))

        t2_key, _ = lax.fori_loop(0, 32, bs2_body, (lo0, hi0))
        t2_f = _float_of_key(t2_key)

        def fill_c2(i, _):
            c2v_v[pl.ds(i * 16, 16)] = ninf16
            c2i_v[pl.ds(i * 16, 16)] = jnp.zeros((16,), jnp.int32)
            return 0

        lax.fori_loop(0, (C2CAP + 16) // 16, fill_c2, 0)

        def rc_body(i, ce2):
            v = candv_v[pl.ds(i * 16, 16)]
            ix = candi_v[pl.ds(i * 16, 16)]
            selm = (v >= t2_f) & (ce2 < C2CAP)
            plsc.store_compressed(c2v_v.at[pl.ds(ce2, 16)], v, mask=selm)
            plsc.store_compressed(c2i_v.at[pl.ds(ce2, 16)], ix, mask=selm)
            return ce2 + jnp.sum(jnp.where(selm, 1, 0).astype(jnp.int32))

        ce2_cnt = lax.fori_loop(0, n_cvr, rc_body, jnp.int32(0))
        n2_cvr = (jnp.minimum(ce2_cnt, C2CAP) + 15) // 16

        def fill_out(i, _):
            outv_v[pl.ds(i * 16, 16)] = jnp.zeros((16,), jnp.float32)
            outi_v[pl.ds(i * 16, 16)] = jnp.zeros((16,), jnp.int32)
            return 0

        lax.fori_loop(0, KPAD // 16, fill_out, 0)

        def rank_blk(ib, _):
            vals = c2v_v[pl.ds(ib * 16, 16)]
            idxs = c2i_v[pl.ds(ib * 16, 16)]

            def rank_lane(l, _):
                lsel = lanes == l
                sv = jnp.max(jnp.where(lsel, vals, NEG_INF))
                si = jnp.max(jnp.where(lsel, idxs, IMIN))

                def cnt_body(jb, c):
                    vj = c2v_v[pl.ds(jb * 16, 16)]
                    ij = c2i_v[pl.ds(jb * 16, 16)]
                    beat = (vj > sv) | ((vj == sv) & (ij < si))
                    return c + jnp.where(beat, 1, 0).astype(jnp.int32)

                rank = jnp.sum(lax.fori_loop(0, n2_cvr, cnt_body,
                                             jnp.zeros((16,), jnp.int32)))
                okm = lsel & (rank < K) & (sv > NEG_INF)
                ridx = _splat_i(rank)
                plsc.store_scatter(outv_v, [ridx], _splat_f(sv), mask=okm)
                plsc.store_scatter(outi_v, [ridx], _splat_i(si), mask=okm)
                return 0

            lax.fori_loop(0, 16, rank_lane, 0)
            return 0

        lax.fori_loop(0, n2_cvr, rank_blk, 0)

        def dec_body(i, _):
            r = i * 16 + lanes
            fl = outi_v[pl.ds(i * 16, 16)]
            q = fl // NUM_CLASSES
            labels_v[pl.ds(i * 16, 16)] = fl - q * NUM_CLASSES
            brow4 = (b * NQ + q) * 4
            plsc.store_scatter(qidx4_v, [r * 4], brow4)
            plsc.store_scatter(qidx4_v, [r * 4 + 1], brow4 + 1)
            plsc.store_scatter(qidx4_v, [r * 4 + 2], brow4 + 2)
            plsc.store_scatter(qidx4_v, [r * 4 + 3], brow4 + 3)
            sv = outv_v[pl.ds(i * 16, 16)]
            scores_v[pl.ds(i * 16, 16)] = 1.0 / (1.0 + jnp.exp(-sv))
            return 0

        lax.fori_loop(0, KPAD // 16, dec_body, 0)

        pltpu.async_copy(boxes_hbm.at[qidx4_v], boxg_v, sem).wait()

        def box_body(i, _):
            r4 = (i * 16 + lanes) * 4
            cx = plsc.load_gather(boxg_v, [r4]) * 640.0
            cy = plsc.load_gather(boxg_v, [r4 + 1]) * 640.0
            w_ = plsc.load_gather(boxg_v, [r4 + 2]) * 640.0
            h_ = plsc.load_gather(boxg_v, [r4 + 3]) * 640.0
            plsc.store_scatter(obox_v, [r4], cx - 0.5 * w_)
            plsc.store_scatter(obox_v, [r4 + 1], cy - 0.5 * h_)
            plsc.store_scatter(obox_v, [r4 + 2], cx + 0.5 * w_)
            plsc.store_scatter(obox_v, [r4 + 3], cy + 0.5 * h_)
            return 0

        lax.fori_loop(0, KPAD // 16, box_body, 0)

        pltpu.sync_copy(labels_v, labels_hbm.at[b])
        pltpu.sync_copy(obox_v, oboxes_hbm.at[b])
        pltpu.sync_copy(scores_v, scores_hbm.at[b])


def kernel(samples, pred_logits, pred_boxes):
    B = pred_logits.shape[0]
    logits1d = pred_logits.reshape(B * NCHUNK * CH)
    logits2d = pred_logits.reshape(B * NCHUNK, CH)
    boxes1d = pred_boxes.reshape(B * NQ * 4)

    mesh = plsc.VectorSubcoreMesh(core_axis_name="c", subcore_axis_name="s")
    run = functools.partial(
        pl.kernel, mesh=mesh,
        out_type=[
            jax.ShapeDtypeStruct((B, KPAD), jnp.int32),
            jax.ShapeDtypeStruct((B, KPAD * 4), jnp.float32),
            jax.ShapeDtypeStruct((B, KPAD), jnp.float32),
        ],
        scratch_types=[
            pltpu.VMEM((WIN,), jnp.float32),
            pltpu.VMEM((HPAD,), jnp.float32),
            pltpu.VMEM((NCPAD,), jnp.float32),
            pltpu.VMEM((NSUP,), jnp.float32),
            pltpu.VMEM((NCPAD + 16,), jnp.int32),
            pltpu.VMEM((MSEL, CH), jnp.float32),
            pltpu.VMEM((CCAP + 16,), jnp.float32),
            pltpu.VMEM((CCAP + 16,), jnp.int32),
            pltpu.VMEM((C2CAP + 16,), jnp.float32),
            pltpu.VMEM((C2CAP + 16,), jnp.int32),
            pltpu.VMEM((KPAD,), jnp.float32),
            pltpu.VMEM((KPAD,), jnp.int32),
            pltpu.VMEM((KPAD,), jnp.int32),
            pltpu.VMEM((KPAD,), jnp.float32),
            pltpu.VMEM((KPAD * 4,), jnp.int32),
            pltpu.VMEM((KPAD * 4,), jnp.float32),
            pltpu.VMEM((KPAD * 4,), jnp.float32),
            pltpu.VMEM_SHARED((8 * NCPAD,), jnp.float32),
            pltpu.SemaphoreType.DMA,
        ],
        compiler_params=pltpu.CompilerParams(needs_layout_passes=False),
    )(_sc_body)
    labels, oboxes, scores = run(logits1d, logits2d, boxes1d)
    return (labels[:, :K],
            oboxes.reshape(B, KPAD, 4)[:, :K],
            scores[:, :K])
